# baseline (speedup 1.0000x reference)
import functools

import jax
import jax.numpy as jnp
from jax.experimental import pallas as pl
from jax.experimental.pallas import tpu as pltpu

BM = 256


def _gcn_kernel(a_row_ref, a_col_ref, x_ref, w1_ref, b1_ref, w2_ref, b2_ref,
                out_ref, y1_s, *, n, nblk):
    k = pl.program_id(0)

    @pl.when(k == 0)
    def _():
        y1 = jnp.dot(x_ref[...].astype(jnp.bfloat16),
                     w1_ref[...].astype(jnp.bfloat16),
                     preferred_element_type=jnp.float32) + b1_ref[...]
        y1_s[...] = y1.astype(jnp.bfloat16)

    h = jnp.dot(a_row_ref[...].astype(jnp.bfloat16), y1_s[...],
                preferred_element_type=jnp.float32)
    h = jnp.maximum(h, 0.0)
    y2_k = (jnp.dot(h.astype(jnp.bfloat16), w2_ref[...].astype(jnp.bfloat16),
                    preferred_element_type=jnp.float32) + b2_ref[...]
            ).astype(jnp.bfloat16)

    if n % BM == 0:
        contrib = jnp.dot(a_col_ref[...].astype(jnp.bfloat16), y2_k,
                          preferred_element_type=jnp.float32)
    else:
        nvalid = n - k * BM

        def _full():
            return jnp.dot(a_col_ref[...].astype(jnp.bfloat16), y2_k,
                           preferred_element_type=jnp.float32)

        def _masked():
            col_ids = jax.lax.broadcasted_iota(jnp.int32, (n, BM), 1)
            a_col = jnp.where(col_ids < nvalid, a_col_ref[...], 0.0)
            row_ids = jax.lax.broadcasted_iota(jnp.int32, y2_k.shape, 0)
            y2m = jnp.where(row_ids < nvalid, y2_k, jnp.bfloat16(0))
            return jnp.dot(a_col.astype(jnp.bfloat16), y2m,
                           preferred_element_type=jnp.float32)

        contrib = jax.lax.cond(k == nblk - 1, _masked, _full)

    @pl.when(k == 0)
    def _():
        out_ref[...] = contrib

    @pl.when(k > 0)
    def _():
        out_ref[...] += contrib


@jax.jit
def kernel(x, adj_t, W1, b1, W2, b2):
    n, d_in = x.shape
    d_hid = W1.shape[1]
    d_out = W2.shape[1]
    nblk = pl.cdiv(n, BM)

    b1r = b1.reshape(1, d_hid)
    b2r = b2.reshape(1, d_out)

    out = pl.pallas_call(
        functools.partial(_gcn_kernel, n=n, nblk=nblk),
        grid=(nblk,),
        in_specs=[
            pl.BlockSpec((BM, n), lambda k: (k, 0)),
            pl.BlockSpec((n, BM), lambda k: (0, k)),
            pl.BlockSpec((n, d_in), lambda k: (0, 0)),
            pl.BlockSpec((d_in, d_hid), lambda k: (0, 0)),
            pl.BlockSpec((1, d_hid), lambda k: (0, 0)),
            pl.BlockSpec((d_hid, d_out), lambda k: (0, 0)),
            pl.BlockSpec((1, d_out), lambda k: (0, 0)),
        ],
        out_specs=pl.BlockSpec((n, d_out), lambda k: (0, 0)),
        out_shape=jax.ShapeDtypeStruct((n, d_out), jnp.float32),
        scratch_shapes=[pltpu.VMEM((n, d_hid), jnp.bfloat16)],
        compiler_params=pltpu.CompilerParams(
            dimension_semantics=(pltpu.GridDimensionSemantics.ARBITRARY,),
        ),
    )(adj_t, adj_t, x, W1, b1r, W2, b2r)

    return out


# speedup vs baseline: 1.0832x; 1.0832x over previous
import functools

import jax
import jax.numpy as jnp
from jax.experimental import pallas as pl
from jax.experimental.pallas import tpu as pltpu

BM = 400


def _gcn_kernel(a_ref, x_ref, w1_ref, b1_ref, w2_ref, b2_ref,
                out_ref, y1_s, y2_s):
    p = pl.program_id(0)
    k = pl.program_id(1)

    @pl.when((p == 0) & (k == 0))
    def _():
        y1 = jnp.dot(x_ref[...].astype(jnp.bfloat16),
                     w1_ref[...].astype(jnp.bfloat16),
                     preferred_element_type=jnp.float32) + b1_ref[...]
        y1_s[...] = y1.astype(jnp.bfloat16)

    @pl.when(p == 0)
    def _():
        h = jnp.dot(a_ref[...].astype(jnp.bfloat16), y1_s[...],
                    preferred_element_type=jnp.float32)
        h = jnp.maximum(h, 0.0)
        y2 = jnp.dot(h.astype(jnp.bfloat16),
                     w2_ref[...].astype(jnp.bfloat16),
                     preferred_element_type=jnp.float32) + b2_ref[...]
        y2_s[pl.ds(k * BM, BM), :] = y2.astype(jnp.bfloat16)

    @pl.when(p == 1)
    def _():
        out_ref[...] = jnp.dot(a_ref[...].astype(jnp.bfloat16), y2_s[...],
                               preferred_element_type=jnp.float32)


@jax.jit
def kernel(x, adj_t, W1, b1, W2, b2):
    n, d_in = x.shape
    d_hid = W1.shape[1]
    d_out = W2.shape[1]
    nblk = pl.cdiv(n, BM)

    b1r = b1.reshape(1, d_hid)
    b2r = b2.reshape(1, d_out)

    out = pl.pallas_call(
        _gcn_kernel,
        grid=(2, nblk),
        in_specs=[
            pl.BlockSpec((BM, n), lambda p, k: (k, 0)),
            pl.BlockSpec((n, d_in), lambda p, k: (0, 0)),
            pl.BlockSpec((d_in, d_hid), lambda p, k: (0, 0)),
            pl.BlockSpec((1, d_hid), lambda p, k: (0, 0)),
            pl.BlockSpec((d_hid, d_out), lambda p, k: (0, 0)),
            pl.BlockSpec((1, d_out), lambda p, k: (0, 0)),
        ],
        out_specs=pl.BlockSpec((BM, d_out), lambda p, k: (p * k, 0)),
        out_shape=jax.ShapeDtypeStruct((n, d_out), jnp.float32),
        scratch_shapes=[
            pltpu.VMEM((n, d_hid), jnp.bfloat16),
            pltpu.VMEM((n, d_out), jnp.bfloat16),
        ],
        compiler_params=pltpu.CompilerParams(
            dimension_semantics=(pltpu.GridDimensionSemantics.ARBITRARY,
                                 pltpu.GridDimensionSemantics.ARBITRARY),
        ),
    )(adj_t, x, W1, b1r, W2, b2r)

    return out
